# Initial kernel scaffold; baseline (speedup 1.0000x reference)
import jax
import jax.numpy as jnp
from jax.experimental import pallas as pl


def kernel(x, edge_index, batch, Wl1, bl1, Wr1, Wl2, bl2, Wr2, Wl3, bl3, Wr3, g1, be1, g2, be2, g3, be3, Wm1, bm1, Wm2, bm2):
    raise NotImplementedError("write your pallas kernel here")


# speedup vs baseline: 4.7451x; 4.7451x over previous
import functools

import jax
import jax.numpy as jnp
from jax import lax
from jax.experimental import pallas as pl
from jax.experimental.pallas import tpu as pltpu
from jax.experimental.pallas import tpu_sc as plsc

_N = 10000
_E = 320000
_H = 128
_C = 40
_EPS = 1e-5

_NC = 2
_NS = 16
_NW = _NC * _NS
_EPT = _E // _NW
_CH = 80
_NCHUNK = _EPT // _CH
_RPB = 624
_RTAIL = _N - _NS * _RPB
_DW = 16

def _mesh():
    return plsc.VectorSubcoreMesh(
        core_axis_name="c", subcore_axis_name="s",
        num_cores=_NC, num_subcores=_NS,
    )


def _zero_rows(ref, nrows, width):
    zv = jnp.zeros((16,), jnp.float32)

    def body(i, carry):
        for j in range(width // 16):
            ref[i, pl.ds(j * 16, 16)] = zv
        return carry

    lax.fori_loop(0, nrows, body, 0)


def _fill_ones(ref, nrows, width):
    ov = jnp.ones((16,), jnp.float32)

    def body(i, carry):
        for j in range(width // 16):
            ref[i, pl.ds(j * 16, 16)] = ov
        return carry

    lax.fori_loop(0, nrows, body, 0)


def _stripe_chunks(total, step):
    chunks = [(k * step, step) for k in range(total // step)]
    if total % step:
        chunks.append((total - total % step, total % step))
    return chunks


def _drain_stripe(src_sh, dst_hbm, c, s, chunk_rows):
    r0 = pl.multiple_of(s * _RPB, 8)
    for off, sz in _stripe_chunks(_RPB, chunk_rows):
        pltpu.sync_copy(src_sh.at[pl.ds(r0 + off, sz)],
                        dst_hbm.at[c, pl.ds(r0 + off, sz)])

    @pl.when(s == _NS - 1)
    def _tail():
        pltpu.sync_copy(src_sh.at[pl.ds(_NS * _RPB, _RTAIL)],
                        dst_hbm.at[c, pl.ds(_NS * _RPB, _RTAIL)])


def _zero_stripe(acc_sh, s, zbuf):
    r0 = pl.multiple_of(s * _RPB, 8)
    for off, sz in _stripe_chunks(_RPB, zbuf.shape[0]):
        pltpu.sync_copy(zbuf.at[pl.ds(0, sz)],
                        acc_sh.at[pl.ds(r0 + off, sz)])

    @pl.when(s == _NS - 1)
    def _tail():
        pltpu.sync_copy(zbuf.at[pl.ds(0, _RTAIL)],
                        acc_sh.at[pl.ds(_NS * _RPB, _RTAIL)])


def _sc_deg_body(dst_hbm, deg_hbm, dst_v, zbuf, ones_v, deg_sh, sem):
    del sem
    c = lax.axis_index("c")
    s = lax.axis_index("s")
    t = c * _NS + s

    _zero_rows(zbuf, _CH, _H)
    _fill_ones(ones_v, _CH, _H)
    _zero_stripe(deg_sh, s, zbuf)
    plsc.subcore_barrier()

    e0 = t * _EPT

    def chunk(i, carry):
        base = pl.multiple_of(e0 + i * _CH, 8)
        pltpu.sync_copy(dst_hbm.at[pl.ds(base, _CH)], dst_v)
        pltpu.sync_copy(ones_v, deg_sh.at[dst_v], add=True)
        return carry

    lax.fori_loop(0, _NCHUNK, chunk, 0)
    plsc.subcore_barrier()

    _drain_stripe(deg_sh, deg_hbm, c, s, _CH)


def _sc_agg_body(y_hbm, src_hbm, dst_hbm, out_hbm,
                 src_v, dst_v, rows_v, zbuf, acc_sh, sem):
    c = lax.axis_index("c")
    s = lax.axis_index("s")
    t = c * _NS + s

    _zero_rows(zbuf, _CH, _H)
    _zero_stripe(acc_sh, s, zbuf)
    plsc.subcore_barrier()

    e0 = t * _EPT

    def chunk(i, carry):
        base = pl.multiple_of(e0 + i * _CH, 8)
        pltpu.sync_copy(src_hbm.at[pl.ds(base, _CH)], src_v)
        pltpu.sync_copy(dst_hbm.at[pl.ds(base, _CH)], dst_v)
        pltpu.async_copy(y_hbm.at[src_v], rows_v, sem).wait()
        pltpu.sync_copy(rows_v, acc_sh.at[dst_v], add=True)
        return carry

    lax.fori_loop(0, _NCHUNK, chunk, 0)
    plsc.subcore_barrier()

    _drain_stripe(acc_sh, out_hbm, c, s, _CH)


@functools.cache
def _sc_deg():
    return pl.kernel(
        _sc_deg_body,
        out_type=jax.ShapeDtypeStruct((_NC, _N, _H), jnp.float32),
        mesh=_mesh(),
        scratch_types=[
            pltpu.VMEM((_CH,), jnp.int32),
            pltpu.VMEM((_CH, _H), jnp.float32),
            pltpu.VMEM((_CH, _H), jnp.float32),
            pltpu.VMEM_SHARED((_N, _H), jnp.float32),
            pltpu.SemaphoreType.DMA,
        ],
    )


@functools.cache
def _sc_agg():
    return pl.kernel(
        _sc_agg_body,
        out_type=jax.ShapeDtypeStruct((_NC, _N, _H), jnp.float32),
        mesh=_mesh(),
        scratch_types=[
            pltpu.VMEM((_CH,), jnp.int32),
            pltpu.VMEM((_CH,), jnp.int32),
            pltpu.VMEM((_CH, _H), jnp.float32),
            pltpu.VMEM((_CH, _H), jnp.float32),
            pltpu.VMEM_SHARED((_N, _H), jnp.float32),
            pltpu.SemaphoreType.DMA,
        ],
    )


def _pre_body(x_ref, wl_ref, wr_ref, y_ref, r_ref):
    x = x_ref[...]
    y_ref[...] = jnp.dot(x, wl_ref[...].T, preferred_element_type=jnp.float32)
    r_ref[...] = jnp.dot(x, wr_ref[...].T, preferred_element_type=jnp.float32)


_tc_pre = pl.pallas_call(
    _pre_body,
    out_shape=[
        jax.ShapeDtypeStruct((_N, _H), jnp.float32),
        jax.ShapeDtypeStruct((_N, _H), jnp.float32),
    ],
)


def _combine(s_ref, deg_ref, r_ref, bl_ref, g_ref, be_ref):
    ssum = s_ref[0] + s_ref[1]
    degw = deg_ref[0] + deg_ref[1]
    deg = jnp.sum(degw, axis=1, keepdims=True) * (1.0 / _H)
    agg = ssum / jnp.maximum(deg, 1.0)
    h = agg + bl_ref[...] + r_ref[...]
    mean = jnp.mean(h, axis=0, keepdims=True)
    var = jnp.mean((h - mean) ** 2, axis=0, keepdims=True)
    hn = g_ref[...] * (h - mean) * lax.rsqrt(var + _EPS) + be_ref[...]
    return jnp.maximum(hn, 0.0)


def _mid_body(s_ref, deg_ref, r_ref, bl_ref, g_ref, be_ref, wln_ref, wrn_ref,
              y_ref, rn_ref):
    h = _combine(s_ref, deg_ref, r_ref, bl_ref, g_ref, be_ref)
    y_ref[...] = jnp.dot(h, wln_ref[...].T, preferred_element_type=jnp.float32)
    rn_ref[...] = jnp.dot(h, wrn_ref[...].T, preferred_element_type=jnp.float32)


_tc_mid = pl.pallas_call(
    _mid_body,
    out_shape=[
        jax.ShapeDtypeStruct((_N, _H), jnp.float32),
        jax.ShapeDtypeStruct((_N, _H), jnp.float32),
    ],
)


def _final_body(s_ref, deg_ref, r_ref, bl_ref, g_ref, be_ref,
                wm1_ref, bm1_ref, wm2_ref, bm2_ref, out_ref):
    h = _combine(s_ref, deg_ref, r_ref, bl_ref, g_ref, be_ref)
    z = jnp.dot(h, wm1_ref[...].T, preferred_element_type=jnp.float32)
    z = z + bm1_ref[...]
    z2 = jnp.dot(z, wm2_ref[...].T, preferred_element_type=jnp.float32)
    z2 = z2 + bm2_ref[...]
    m = jnp.max(z2, axis=1, keepdims=True)
    lse = jnp.log(jnp.sum(jnp.exp(z2 - m), axis=1, keepdims=True)) + m
    out_ref[...] = z2 - lse


_tc_final = pl.pallas_call(
    _final_body,
    out_shape=jax.ShapeDtypeStruct((_N, _C), jnp.float32),
)


def kernel(x, edge_index, batch, Wl1, bl1, Wr1, Wl2, bl2, Wr2, Wl3, bl3, Wr3,
           g1, be1, g2, be2, g3, be3, Wm1, bm1, Wm2, bm2):
    del batch
    src = edge_index[0].astype(jnp.int32)
    dst = edge_index[1].astype(jnp.int32)

    bl1r, bl2r, bl3r = (b.reshape(1, _H) for b in (bl1, bl2, bl3))
    g1r, g2r, g3r = (g.reshape(1, _H) for g in (g1, g2, g3))
    be1r, be2r, be3r = (b.reshape(1, _H) for b in (be1, be2, be3))
    bm1r = bm1.reshape(1, _H)
    bm2r = bm2.reshape(1, _C)

    y1, r1 = _tc_pre(x, Wl1, Wr1)
    deg = _sc_deg()(dst)
    s1 = _sc_agg()(y1, src, dst)
    y2, r2 = _tc_mid(s1, deg, r1, bl1r, g1r, be1r, Wl2, Wr2)
    s2 = _sc_agg()(y2, src, dst)
    y3, r3 = _tc_mid(s2, deg, r2, bl2r, g2r, be2r, Wl3, Wr3)
    s3 = _sc_agg()(y3, src, dst)
    out = _tc_final(s3, deg, r3, bl3r, g3r, be3r, Wm1, bm1r, Wm2, bm2r)
    return out


# speedup vs baseline: 8.5876x; 1.8098x over previous
import functools

import jax
import jax.numpy as jnp
from jax import lax
from jax.experimental import pallas as pl
from jax.experimental.pallas import tpu as pltpu
from jax.experimental.pallas import tpu_sc as plsc

_N = 10000
_E = 320000
_H = 128
_C = 40
_EPS = 1e-5

_NC = 2
_NS = 16
_NW = _NC * _NS
_EPT = _E // _NW
_CH = 80
_NCHUNK = _EPT // _CH
_RPB = 624
_RTAIL = _N - _NS * _RPB
_DW = 16

def _mesh():
    return plsc.VectorSubcoreMesh(
        core_axis_name="c", subcore_axis_name="s",
        num_cores=_NC, num_subcores=_NS,
    )


def _zero_rows(ref, nrows, width):
    zv = jnp.zeros((16,), jnp.float32)

    def body(i, carry):
        for j in range(width // 16):
            ref[i, pl.ds(j * 16, 16)] = zv
        return carry

    lax.fori_loop(0, nrows, body, 0)


def _fill_ones(ref, nrows, width):
    ov = jnp.ones((16,), jnp.float32)

    def body(i, carry):
        for j in range(width // 16):
            ref[i, pl.ds(j * 16, 16)] = ov
        return carry

    lax.fori_loop(0, nrows, body, 0)


def _stripe_chunks(total, step):
    chunks = [(k * step, step) for k in range(total // step)]
    if total % step:
        chunks.append((total - total % step, total % step))
    return chunks


def _drain_stripe(src_sh, dst_hbm, c, s, chunk_rows):
    r0 = pl.multiple_of(s * _RPB, 8)
    for off, sz in _stripe_chunks(_RPB, chunk_rows):
        pltpu.sync_copy(src_sh.at[pl.ds(r0 + off, sz)],
                        dst_hbm.at[c, pl.ds(r0 + off, sz)])

    @pl.when(s == _NS - 1)
    def _tail():
        pltpu.sync_copy(src_sh.at[pl.ds(_NS * _RPB, _RTAIL)],
                        dst_hbm.at[c, pl.ds(_NS * _RPB, _RTAIL)])


def _zero_stripe(acc_sh, s, zbuf):
    r0 = pl.multiple_of(s * _RPB, 8)
    for off, sz in _stripe_chunks(_RPB, zbuf.shape[0]):
        pltpu.sync_copy(zbuf.at[pl.ds(0, sz)],
                        acc_sh.at[pl.ds(r0 + off, sz)])

    @pl.when(s == _NS - 1)
    def _tail():
        pltpu.sync_copy(zbuf.at[pl.ds(0, _RTAIL)],
                        acc_sh.at[pl.ds(_NS * _RPB, _RTAIL)])


_DNBUF = 5
_NBUF = 4
_NTAIL = _NCHUNK - (_NCHUNK // _NBUF) * _NBUF


def _sc_deg_body(dst_hbm, deg_hbm, dst_vs, ones_v, zbuf, deg_sh, isems,
                 ssems):
    c = lax.axis_index("c")
    s = lax.axis_index("s")
    t = c * _NS + s

    _zero_rows(zbuf, _CH, _H)
    _fill_ones(ones_v, _CH, _H)
    _zero_stripe(deg_sh, s, zbuf)
    plsc.subcore_barrier()

    e0 = t * _EPT

    def group(i, carry):
        j0 = i * _DNBUF
        ids, sds = [], []
        for b in range(_DNBUF):
            base = pl.multiple_of(e0 + (j0 + b) * _CH, 8)
            ids.append(pltpu.async_copy(dst_hbm.at[pl.ds(base, _CH)],
                                        dst_vs[b], isems[b]))
        for b in range(_DNBUF):
            ids[b].wait()
            sds.append(pltpu.async_copy(ones_v, deg_sh.at[dst_vs[b]],
                                        ssems[b], add=True))
        for b in range(_DNBUF):
            sds[b].wait()
        return carry

    lax.fori_loop(0, _NCHUNK // _DNBUF, group, 0)
    plsc.subcore_barrier()

    _drain_stripe(deg_sh, deg_hbm, c, s, _CH)


def _sc_agg_body(y_hbm, src_hbm, dst_hbm, out_hbm,
                 src_vs, dst_vs, rows, acc_sh,
                 isems, gsems, ssems):
    c = lax.axis_index("c")
    s = lax.axis_index("s")
    t = c * _NS + s

    _zero_rows(rows[0], _CH, _H)
    _zero_stripe(acc_sh, s, rows[0])
    plsc.subcore_barrier()

    e0 = t * _EPT

    def do_chunks(j0, nbuf):
        ids, gds, sds = [], [], []
        for b in range(nbuf):
            base = pl.multiple_of(e0 + (j0 + b) * _CH, 8)
            i0 = pltpu.async_copy(src_hbm.at[pl.ds(base, _CH)], src_vs[b],
                                  isems[b])
            i1 = pltpu.async_copy(dst_hbm.at[pl.ds(base, _CH)], dst_vs[b],
                                  isems[b])
            ids.append((i0, i1))
        for b in range(nbuf):
            ids[b][0].wait()
            gds.append(pltpu.async_copy(y_hbm.at[src_vs[b]], rows[b],
                                        gsems[b]))
        for b in range(nbuf):
            gds[b].wait()
            ids[b][1].wait()
            sds.append(pltpu.async_copy(rows[b], acc_sh.at[dst_vs[b]],
                                        ssems[b], add=True))
        for b in range(nbuf):
            sds[b].wait()

    def group(i, carry):
        do_chunks(i * _NBUF, _NBUF)
        return carry

    lax.fori_loop(0, _NCHUNK // _NBUF, group, 0)
    if _NTAIL:
        do_chunks((_NCHUNK // _NBUF) * _NBUF, _NTAIL)
    plsc.subcore_barrier()

    _drain_stripe(acc_sh, out_hbm, c, s, _CH)


@functools.cache
def _sc_deg():
    return pl.kernel(
        _sc_deg_body,
        out_type=jax.ShapeDtypeStruct((_NC, _N, _H), jnp.float32),
        mesh=_mesh(),
        scratch_types=[
            [pltpu.VMEM((_CH,), jnp.int32) for _ in range(_DNBUF)],
            pltpu.VMEM((_CH, _H), jnp.float32),
            pltpu.VMEM((_CH, _H), jnp.float32),
            pltpu.VMEM_SHARED((_N, _H), jnp.float32),
            [pltpu.SemaphoreType.DMA for _ in range(_DNBUF)],
            [pltpu.SemaphoreType.DMA for _ in range(_DNBUF)],
        ],
    )


@functools.cache
def _sc_agg():
    return pl.kernel(
        _sc_agg_body,
        out_type=jax.ShapeDtypeStruct((_NC, _N, _H), jnp.float32),
        mesh=_mesh(),
        scratch_types=[
            [pltpu.VMEM((_CH,), jnp.int32) for _ in range(_NBUF)],
            [pltpu.VMEM((_CH,), jnp.int32) for _ in range(_NBUF)],
            [pltpu.VMEM((_CH, _H), jnp.float32) for _ in range(_NBUF)],
            pltpu.VMEM_SHARED((_N, _H), jnp.float32),
            [pltpu.SemaphoreType.DMA for _ in range(_NBUF)],
            [pltpu.SemaphoreType.DMA for _ in range(_NBUF)],
            [pltpu.SemaphoreType.DMA for _ in range(_NBUF)],
        ],
    )


def _pre_body(x_ref, wl_ref, wr_ref, y_ref, r_ref):
    x = x_ref[...]
    y_ref[...] = jnp.dot(x, wl_ref[...].T, preferred_element_type=jnp.float32)
    r_ref[...] = jnp.dot(x, wr_ref[...].T, preferred_element_type=jnp.float32)


_tc_pre = pl.pallas_call(
    _pre_body,
    out_shape=[
        jax.ShapeDtypeStruct((_N, _H), jnp.float32),
        jax.ShapeDtypeStruct((_N, _H), jnp.float32),
    ],
)


def _combine(s_ref, deg_ref, r_ref, bl_ref, g_ref, be_ref):
    ssum = s_ref[0] + s_ref[1]
    degw = deg_ref[0] + deg_ref[1]
    deg = jnp.sum(degw, axis=1, keepdims=True) * (1.0 / _H)
    agg = ssum / jnp.maximum(deg, 1.0)
    h = agg + bl_ref[...] + r_ref[...]
    mean = jnp.mean(h, axis=0, keepdims=True)
    var = jnp.mean((h - mean) ** 2, axis=0, keepdims=True)
    hn = g_ref[...] * (h - mean) * lax.rsqrt(var + _EPS) + be_ref[...]
    return jnp.maximum(hn, 0.0)


def _mid_body(s_ref, deg_ref, r_ref, bl_ref, g_ref, be_ref, wln_ref, wrn_ref,
              y_ref, rn_ref):
    h = _combine(s_ref, deg_ref, r_ref, bl_ref, g_ref, be_ref)
    y_ref[...] = jnp.dot(h, wln_ref[...].T, preferred_element_type=jnp.float32)
    rn_ref[...] = jnp.dot(h, wrn_ref[...].T, preferred_element_type=jnp.float32)


_tc_mid = pl.pallas_call(
    _mid_body,
    out_shape=[
        jax.ShapeDtypeStruct((_N, _H), jnp.float32),
        jax.ShapeDtypeStruct((_N, _H), jnp.float32),
    ],
)


def _final_body(s_ref, deg_ref, r_ref, bl_ref, g_ref, be_ref,
                wm1_ref, bm1_ref, wm2_ref, bm2_ref, out_ref):
    h = _combine(s_ref, deg_ref, r_ref, bl_ref, g_ref, be_ref)
    z = jnp.dot(h, wm1_ref[...].T, preferred_element_type=jnp.float32)
    z = z + bm1_ref[...]
    z2 = jnp.dot(z, wm2_ref[...].T, preferred_element_type=jnp.float32)
    z2 = z2 + bm2_ref[...]
    m = jnp.max(z2, axis=1, keepdims=True)
    lse = jnp.log(jnp.sum(jnp.exp(z2 - m), axis=1, keepdims=True)) + m
    out_ref[...] = z2 - lse


_tc_final = pl.pallas_call(
    _final_body,
    out_shape=jax.ShapeDtypeStruct((_N, _C), jnp.float32),
)


def kernel(x, edge_index, batch, Wl1, bl1, Wr1, Wl2, bl2, Wr2, Wl3, bl3, Wr3,
           g1, be1, g2, be2, g3, be3, Wm1, bm1, Wm2, bm2):
    del batch
    src = edge_index[0].astype(jnp.int32)
    dst = edge_index[1].astype(jnp.int32)

    bl1r, bl2r, bl3r = (b.reshape(1, _H) for b in (bl1, bl2, bl3))
    g1r, g2r, g3r = (g.reshape(1, _H) for g in (g1, g2, g3))
    be1r, be2r, be3r = (b.reshape(1, _H) for b in (be1, be2, be3))
    bm1r = bm1.reshape(1, _H)
    bm2r = bm2.reshape(1, _C)

    y1, r1 = _tc_pre(x, Wl1, Wr1)
    deg = _sc_deg()(dst)
    s1 = _sc_agg()(y1, src, dst)
    y2, r2 = _tc_mid(s1, deg, r1, bl1r, g1r, be1r, Wl2, Wr2)
    s2 = _sc_agg()(y2, src, dst)
    y3, r3 = _tc_mid(s2, deg, r2, bl2r, g2r, be2r, Wl3, Wr3)
    s3 = _sc_agg()(y3, src, dst)
    out = _tc_final(s3, deg, r3, bl3r, g3r, be3r, Wm1, bm1r, Wm2, bm2r)
    return out


# speedup vs baseline: 9.5144x; 1.1079x over previous
import functools

import jax
import jax.numpy as jnp
from jax import lax
from jax.experimental import pallas as pl
from jax.experimental.pallas import tpu as pltpu
from jax.experimental.pallas import tpu_sc as plsc

_N = 10000
_E = 320000
_H = 128
_C = 40
_EPS = 1e-5

_NC = 2
_NS = 16
_NW = _NC * _NS
_EPT = _E // _NW
_CH = 80
_NCHUNK = _EPT // _CH
_RPB = 624
_RTAIL = _N - _NS * _RPB
_DW = 16

def _mesh():
    return plsc.VectorSubcoreMesh(
        core_axis_name="c", subcore_axis_name="s",
        num_cores=_NC, num_subcores=_NS,
    )


def _zero_rows(ref, nrows, width):
    zv = jnp.zeros((16,), jnp.float32)

    def body(i, carry):
        for j in range(width // 16):
            ref[i, pl.ds(j * 16, 16)] = zv
        return carry

    lax.fori_loop(0, nrows, body, 0)


def _fill_ones(ref, nrows, width):
    ov = jnp.ones((16,), jnp.float32)

    def body(i, carry):
        for j in range(width // 16):
            ref[i, pl.ds(j * 16, 16)] = ov
        return carry

    lax.fori_loop(0, nrows, body, 0)


def _stripe_chunks(total, step):
    chunks = [(k * step, step) for k in range(total // step)]
    if total % step:
        chunks.append((total - total % step, total % step))
    return chunks


def _drain_stripe(src_sh, dst_hbm, c, s, chunk_rows):
    r0 = pl.multiple_of(s * _RPB, 8)
    for off, sz in _stripe_chunks(_RPB, chunk_rows):
        pltpu.sync_copy(src_sh.at[pl.ds(r0 + off, sz)],
                        dst_hbm.at[c, pl.ds(r0 + off, sz)])

    @pl.when(s == _NS - 1)
    def _tail():
        pltpu.sync_copy(src_sh.at[pl.ds(_NS * _RPB, _RTAIL)],
                        dst_hbm.at[c, pl.ds(_NS * _RPB, _RTAIL)])


def _zero_stripe(acc_sh, s, zbuf):
    r0 = pl.multiple_of(s * _RPB, 8)
    for off, sz in _stripe_chunks(_RPB, zbuf.shape[0]):
        pltpu.sync_copy(zbuf.at[pl.ds(0, sz)],
                        acc_sh.at[pl.ds(r0 + off, sz)])

    @pl.when(s == _NS - 1)
    def _tail():
        pltpu.sync_copy(zbuf.at[pl.ds(0, _RTAIL)],
                        acc_sh.at[pl.ds(_NS * _RPB, _RTAIL)])


_DNBUF = 5
_NBUF = 4
_NTAIL = _NCHUNK - (_NCHUNK // _NBUF) * _NBUF


def _sc_deg_body(dst_hbm, deg_hbm, dst_vs, ones_v, zbuf, deg_sh, isems,
                 ssems):
    c = lax.axis_index("c")
    s = lax.axis_index("s")
    t = c * _NS + s

    _zero_rows(zbuf, _CH, _H)
    _fill_ones(ones_v, _CH, _H)
    _zero_stripe(deg_sh, s, zbuf)
    plsc.subcore_barrier()

    e0 = t * _EPT

    def wait_scatter(b):
        pltpu.make_async_copy(ones_v, deg_sh.at[dst_vs[b]], ssems[b]).wait()

    def group(i, carry):
        j0 = i * _DNBUF
        ids = []
        for b in range(_DNBUF):
            @pl.when(i > 0)
            def _w(b=b):
                wait_scatter(b)

            base = pl.multiple_of(e0 + (j0 + b) * _CH, 8)
            ids.append(pltpu.async_copy(dst_hbm.at[pl.ds(base, _CH)],
                                        dst_vs[b], isems[b]))
        for b in range(_DNBUF):
            ids[b].wait()
            pltpu.async_copy(ones_v, deg_sh.at[dst_vs[b]], ssems[b],
                             add=True)
        return carry

    lax.fori_loop(0, _NCHUNK // _DNBUF, group, 0)
    for b in range(_DNBUF):
        wait_scatter(b)
    plsc.subcore_barrier()

    _drain_stripe(deg_sh, deg_hbm, c, s, _CH)


def _sc_agg_body(y_hbm, src_hbm, dst_hbm, out_hbm,
                 src_vs, dst_vs, rows, acc_sh,
                 isems, gsems, ssems):
    c = lax.axis_index("c")
    s = lax.axis_index("s")
    t = c * _NS + s

    _zero_rows(rows[0], _CH, _H)
    _zero_stripe(acc_sh, s, rows[0])
    plsc.subcore_barrier()

    e0 = t * _EPT

    def wait_scatter(b):
        pltpu.make_async_copy(rows[b], acc_sh.at[dst_vs[b]], ssems[b]).wait()

    def group(i, carry):
        j0 = i * _NBUF
        ids, gds = [], []
        for b in range(_NBUF):
            @pl.when(i > 0)
            def _w(b=b):
                wait_scatter(b)

            base = pl.multiple_of(e0 + (j0 + b) * _CH, 8)
            i0 = pltpu.async_copy(src_hbm.at[pl.ds(base, _CH)], src_vs[b],
                                  isems[b])
            i1 = pltpu.async_copy(dst_hbm.at[pl.ds(base, _CH)], dst_vs[b],
                                  isems[b])
            ids.append((i0, i1))
        for b in range(_NBUF):
            ids[b][0].wait()
            gds.append(pltpu.async_copy(y_hbm.at[src_vs[b]], rows[b],
                                        gsems[b]))
        for b in range(_NBUF):
            gds[b].wait()
            ids[b][1].wait()
            pltpu.async_copy(rows[b], acc_sh.at[dst_vs[b]], ssems[b],
                             add=True)
        return carry

    ngroups = _NCHUNK // _NBUF
    lax.fori_loop(0, ngroups, group, 0)
    for j in range(ngroups * _NBUF, _NCHUNK):
        wait_scatter(0)
        base = pl.multiple_of(e0 + j * _CH, 8)
        i0 = pltpu.async_copy(src_hbm.at[pl.ds(base, _CH)], src_vs[0],
                              isems[0])
        i1 = pltpu.async_copy(dst_hbm.at[pl.ds(base, _CH)], dst_vs[0],
                              isems[0])
        i0.wait()
        pltpu.async_copy(y_hbm.at[src_vs[0]], rows[0], gsems[0]).wait()
        i1.wait()
        pltpu.async_copy(rows[0], acc_sh.at[dst_vs[0]], ssems[0], add=True)
    for b in range(_NBUF):
        wait_scatter(b)
    plsc.subcore_barrier()

    _drain_stripe(acc_sh, out_hbm, c, s, _CH)


@functools.cache
def _sc_deg():
    return pl.kernel(
        _sc_deg_body,
        out_type=jax.ShapeDtypeStruct((_NC, _N, _H), jnp.float32),
        mesh=_mesh(),
        scratch_types=[
            [pltpu.VMEM((_CH,), jnp.int32) for _ in range(_DNBUF)],
            pltpu.VMEM((_CH, _H), jnp.float32),
            pltpu.VMEM((_CH, _H), jnp.float32),
            pltpu.VMEM_SHARED((_N, _H), jnp.float32),
            [pltpu.SemaphoreType.DMA for _ in range(_DNBUF)],
            [pltpu.SemaphoreType.DMA for _ in range(_DNBUF)],
        ],
    )


@functools.cache
def _sc_agg():
    return pl.kernel(
        _sc_agg_body,
        out_type=jax.ShapeDtypeStruct((_NC, _N, _H), jnp.float32),
        mesh=_mesh(),
        scratch_types=[
            [pltpu.VMEM((_CH,), jnp.int32) for _ in range(_NBUF)],
            [pltpu.VMEM((_CH,), jnp.int32) for _ in range(_NBUF)],
            [pltpu.VMEM((_CH, _H), jnp.float32) for _ in range(_NBUF)],
            pltpu.VMEM_SHARED((_N, _H), jnp.float32),
            [pltpu.SemaphoreType.DMA for _ in range(_NBUF)],
            [pltpu.SemaphoreType.DMA for _ in range(_NBUF)],
            [pltpu.SemaphoreType.DMA for _ in range(_NBUF)],
        ],
    )


def _pre_body(x_ref, wl_ref, wr_ref, y_ref, r_ref):
    x = x_ref[...]
    y_ref[...] = jnp.dot(x, wl_ref[...].T, preferred_element_type=jnp.float32)
    r_ref[...] = jnp.dot(x, wr_ref[...].T, preferred_element_type=jnp.float32)


_tc_pre = pl.pallas_call(
    _pre_body,
    out_shape=[
        jax.ShapeDtypeStruct((_N, _H), jnp.float32),
        jax.ShapeDtypeStruct((_N, _H), jnp.float32),
    ],
)


def _combine(s_ref, deg_ref, r_ref, bl_ref, g_ref, be_ref):
    ssum = s_ref[0] + s_ref[1]
    degw = deg_ref[0] + deg_ref[1]
    deg = jnp.sum(degw, axis=1, keepdims=True) * (1.0 / _H)
    agg = ssum / jnp.maximum(deg, 1.0)
    h = agg + bl_ref[...] + r_ref[...]
    mean = jnp.mean(h, axis=0, keepdims=True)
    var = jnp.mean((h - mean) ** 2, axis=0, keepdims=True)
    hn = g_ref[...] * (h - mean) * lax.rsqrt(var + _EPS) + be_ref[...]
    return jnp.maximum(hn, 0.0)


def _mid_body(s_ref, deg_ref, r_ref, bl_ref, g_ref, be_ref, wln_ref, wrn_ref,
              y_ref, rn_ref):
    h = _combine(s_ref, deg_ref, r_ref, bl_ref, g_ref, be_ref)
    y_ref[...] = jnp.dot(h, wln_ref[...].T, preferred_element_type=jnp.float32)
    rn_ref[...] = jnp.dot(h, wrn_ref[...].T, preferred_element_type=jnp.float32)


_tc_mid = pl.pallas_call(
    _mid_body,
    out_shape=[
        jax.ShapeDtypeStruct((_N, _H), jnp.float32),
        jax.ShapeDtypeStruct((_N, _H), jnp.float32),
    ],
)


def _final_body(s_ref, deg_ref, r_ref, bl_ref, g_ref, be_ref,
                wm1_ref, bm1_ref, wm2_ref, bm2_ref, out_ref):
    h = _combine(s_ref, deg_ref, r_ref, bl_ref, g_ref, be_ref)
    z = jnp.dot(h, wm1_ref[...].T, preferred_element_type=jnp.float32)
    z = z + bm1_ref[...]
    z2 = jnp.dot(z, wm2_ref[...].T, preferred_element_type=jnp.float32)
    z2 = z2 + bm2_ref[...]
    m = jnp.max(z2, axis=1, keepdims=True)
    lse = jnp.log(jnp.sum(jnp.exp(z2 - m), axis=1, keepdims=True)) + m
    out_ref[...] = z2 - lse


_tc_final = pl.pallas_call(
    _final_body,
    out_shape=jax.ShapeDtypeStruct((_N, _C), jnp.float32),
)


def kernel(x, edge_index, batch, Wl1, bl1, Wr1, Wl2, bl2, Wr2, Wl3, bl3, Wr3,
           g1, be1, g2, be2, g3, be3, Wm1, bm1, Wm2, bm2):
    del batch
    src = edge_index[0].astype(jnp.int32)
    dst = edge_index[1].astype(jnp.int32)

    bl1r, bl2r, bl3r = (b.reshape(1, _H) for b in (bl1, bl2, bl3))
    g1r, g2r, g3r = (g.reshape(1, _H) for g in (g1, g2, g3))
    be1r, be2r, be3r = (b.reshape(1, _H) for b in (be1, be2, be3))
    bm1r = bm1.reshape(1, _H)
    bm2r = bm2.reshape(1, _C)

    y1, r1 = _tc_pre(x, Wl1, Wr1)
    deg = _sc_deg()(dst)
    s1 = _sc_agg()(y1, src, dst)
    y2, r2 = _tc_mid(s1, deg, r1, bl1r, g1r, be1r, Wl2, Wr2)
    s2 = _sc_agg()(y2, src, dst)
    y3, r3 = _tc_mid(s2, deg, r2, bl2r, g2r, be2r, Wl3, Wr3)
    s3 = _sc_agg()(y3, src, dst)
    out = _tc_final(s3, deg, r3, bl3r, g3r, be3r, Wm1, bm1r, Wm2, bm2r)
    return out


# speedup vs baseline: 9.5883x; 1.0078x over previous
import functools

import jax
import jax.numpy as jnp
from jax import lax
from jax.experimental import pallas as pl
from jax.experimental.pallas import tpu as pltpu
from jax.experimental.pallas import tpu_sc as plsc

_N = 10000
_E = 320000
_H = 128
_C = 40
_EPS = 1e-5

_NC = 2
_NS = 16
_NW = _NC * _NS
_EPT = _E // _NW
_CH = 96
_NCHUNK = _EPT // _CH
_CHT = _EPT - _NCHUNK * _CH
_RPB = 624
_RTAIL = _N - _NS * _RPB
_DW = 16

def _mesh():
    return plsc.VectorSubcoreMesh(
        core_axis_name="c", subcore_axis_name="s",
        num_cores=_NC, num_subcores=_NS,
    )


def _zero_rows(ref, nrows, width):
    zv = jnp.zeros((16,), jnp.float32)

    def body(i, carry):
        for j in range(width // 16):
            ref[i, pl.ds(j * 16, 16)] = zv
        return carry

    lax.fori_loop(0, nrows, body, 0)


def _fill_ones(ref, nrows, width):
    ov = jnp.ones((16,), jnp.float32)

    def body(i, carry):
        for j in range(width // 16):
            ref[i, pl.ds(j * 16, 16)] = ov
        return carry

    lax.fori_loop(0, nrows, body, 0)


def _stripe_chunks(total, step):
    chunks = [(k * step, step) for k in range(total // step)]
    if total % step:
        chunks.append((total - total % step, total % step))
    return chunks


def _drain_stripe(src_sh, dst_hbm, c, s, chunk_rows):
    r0 = pl.multiple_of(s * _RPB, 8)
    for off, sz in _stripe_chunks(_RPB, chunk_rows):
        pltpu.sync_copy(src_sh.at[pl.ds(r0 + off, sz)],
                        dst_hbm.at[c, pl.ds(r0 + off, sz)])

    @pl.when(s == _NS - 1)
    def _tail():
        pltpu.sync_copy(src_sh.at[pl.ds(_NS * _RPB, _RTAIL)],
                        dst_hbm.at[c, pl.ds(_NS * _RPB, _RTAIL)])


def _zero_stripe(acc_sh, s, zbuf):
    r0 = pl.multiple_of(s * _RPB, 8)
    for off, sz in _stripe_chunks(_RPB, zbuf.shape[0]):
        pltpu.sync_copy(zbuf.at[pl.ds(0, sz)],
                        acc_sh.at[pl.ds(r0 + off, sz)])

    @pl.when(s == _NS - 1)
    def _tail():
        pltpu.sync_copy(zbuf.at[pl.ds(0, _RTAIL)],
                        acc_sh.at[pl.ds(_NS * _RPB, _RTAIL)])


_DNBUF = 8
_NBUF = 4
assert _NCHUNK % _NBUF == 0 and _NCHUNK % _DNBUF == 0


def _sc_deg_body(dst_hbm, deg_hbm, dst_vs, dtail_v, ones_v, zbuf, deg_sh,
                 isems, ssems):
    c = lax.axis_index("c")
    s = lax.axis_index("s")
    t = c * _NS + s

    _zero_rows(zbuf, _CH, _H)
    _fill_ones(ones_v, _CH, _H)
    _zero_stripe(deg_sh, s, zbuf)
    plsc.subcore_barrier()

    e0 = t * _EPT

    def wait_scatter(b):
        pltpu.make_async_copy(ones_v, deg_sh.at[dst_vs[b]], ssems[b]).wait()

    def group(i, carry):
        j0 = i * _DNBUF
        ids = []
        for b in range(_DNBUF):
            @pl.when(i > 0)
            def _w(b=b):
                wait_scatter(b)

            base = pl.multiple_of(e0 + (j0 + b) * _CH, 8)
            ids.append(pltpu.async_copy(dst_hbm.at[pl.ds(base, _CH)],
                                        dst_vs[b], isems[b]))
        for b in range(_DNBUF):
            ids[b].wait()
            pltpu.async_copy(ones_v, deg_sh.at[dst_vs[b]], ssems[b],
                             add=True)
        return carry

    lax.fori_loop(0, _NCHUNK // _DNBUF, group, 0)
    for b in range(_DNBUF):
        wait_scatter(b)
    base = pl.multiple_of(e0 + _NCHUNK * _CH, 8)
    ones_t = ones_v.at[pl.ds(0, _CHT)]
    pltpu.async_copy(dst_hbm.at[pl.ds(base, _CHT)], dtail_v, isems[0]).wait()
    pltpu.async_copy(ones_t, deg_sh.at[dtail_v], ssems[0], add=True).wait()
    plsc.subcore_barrier()

    _drain_stripe(deg_sh, deg_hbm, c, s, _CH)


def _sc_agg_body(y_hbm, src_hbm, dst_hbm, out_hbm,
                 src_vs, dst_vs, rows, stail_v, dtail_v, acc_sh,
                 isems, gsems, ssems):
    c = lax.axis_index("c")
    s = lax.axis_index("s")
    t = c * _NS + s

    _zero_rows(rows[0], _CH, _H)
    _zero_stripe(acc_sh, s, rows[0])
    plsc.subcore_barrier()

    e0 = t * _EPT

    def wait_scatter(b):
        pltpu.make_async_copy(rows[b], acc_sh.at[dst_vs[b]], ssems[b]).wait()

    def group(i, carry):
        j0 = i * _NBUF
        ids, gds = [], []
        for b in range(_NBUF):
            @pl.when(i > 0)
            def _w(b=b):
                wait_scatter(b)

            base = pl.multiple_of(e0 + (j0 + b) * _CH, 8)
            i0 = pltpu.async_copy(src_hbm.at[pl.ds(base, _CH)], src_vs[b],
                                  isems[b])
            i1 = pltpu.async_copy(dst_hbm.at[pl.ds(base, _CH)], dst_vs[b],
                                  isems[b])
            ids.append((i0, i1))
        for b in range(_NBUF):
            ids[b][0].wait()
            gds.append(pltpu.async_copy(y_hbm.at[src_vs[b]], rows[b],
                                        gsems[b]))
        for b in range(_NBUF):
            gds[b].wait()
            ids[b][1].wait()
            pltpu.async_copy(rows[b], acc_sh.at[dst_vs[b]], ssems[b],
                             add=True)
        return carry

    lax.fori_loop(0, _NCHUNK // _NBUF, group, 0)
    for b in range(_NBUF):
        wait_scatter(b)
    base = pl.multiple_of(e0 + _NCHUNK * _CH, 8)
    rows_t = rows[0].at[pl.ds(0, _CHT)]
    i0 = pltpu.async_copy(src_hbm.at[pl.ds(base, _CHT)], stail_v, isems[0])
    i1 = pltpu.async_copy(dst_hbm.at[pl.ds(base, _CHT)], dtail_v, isems[0])
    i0.wait()
    pltpu.async_copy(y_hbm.at[stail_v], rows_t, gsems[0]).wait()
    i1.wait()
    pltpu.async_copy(rows_t, acc_sh.at[dtail_v], ssems[0], add=True).wait()
    plsc.subcore_barrier()

    _drain_stripe(acc_sh, out_hbm, c, s, _CH)


@functools.cache
def _sc_deg():
    return pl.kernel(
        _sc_deg_body,
        out_type=jax.ShapeDtypeStruct((_NC, _N, _H), jnp.float32),
        mesh=_mesh(),
        scratch_types=[
            [pltpu.VMEM((_CH,), jnp.int32) for _ in range(_DNBUF)],
            pltpu.VMEM((_CHT,), jnp.int32),
            pltpu.VMEM((_CH, _H), jnp.float32),
            pltpu.VMEM((_CH, _H), jnp.float32),
            pltpu.VMEM_SHARED((_N, _H), jnp.float32),
            [pltpu.SemaphoreType.DMA for _ in range(_DNBUF)],
            [pltpu.SemaphoreType.DMA for _ in range(_DNBUF)],
        ],
    )


@functools.cache
def _sc_agg():
    return pl.kernel(
        _sc_agg_body,
        out_type=jax.ShapeDtypeStruct((_NC, _N, _H), jnp.float32),
        mesh=_mesh(),
        scratch_types=[
            [pltpu.VMEM((_CH,), jnp.int32) for _ in range(_NBUF)],
            [pltpu.VMEM((_CH,), jnp.int32) for _ in range(_NBUF)],
            [pltpu.VMEM((_CH, _H), jnp.float32) for _ in range(_NBUF)],
            pltpu.VMEM((_CHT,), jnp.int32),
            pltpu.VMEM((_CHT,), jnp.int32),
            pltpu.VMEM_SHARED((_N, _H), jnp.float32),
            [pltpu.SemaphoreType.DMA for _ in range(_NBUF)],
            [pltpu.SemaphoreType.DMA for _ in range(_NBUF)],
            [pltpu.SemaphoreType.DMA for _ in range(_NBUF)],
        ],
    )


def _pre_body(x_ref, wl_ref, wr_ref, y_ref, r_ref):
    x = x_ref[...]
    y_ref[...] = jnp.dot(x, wl_ref[...].T, preferred_element_type=jnp.float32)
    r_ref[...] = jnp.dot(x, wr_ref[...].T, preferred_element_type=jnp.float32)


_tc_pre = pl.pallas_call(
    _pre_body,
    out_shape=[
        jax.ShapeDtypeStruct((_N, _H), jnp.float32),
        jax.ShapeDtypeStruct((_N, _H), jnp.float32),
    ],
)


def _combine(s_ref, deg_ref, r_ref, bl_ref, g_ref, be_ref):
    ssum = s_ref[0] + s_ref[1]
    degw = deg_ref[0] + deg_ref[1]
    deg = jnp.sum(degw, axis=1, keepdims=True) * (1.0 / _H)
    agg = ssum / jnp.maximum(deg, 1.0)
    h = agg + bl_ref[...] + r_ref[...]
    mean = jnp.mean(h, axis=0, keepdims=True)
    var = jnp.mean((h - mean) ** 2, axis=0, keepdims=True)
    hn = g_ref[...] * (h - mean) * lax.rsqrt(var + _EPS) + be_ref[...]
    return jnp.maximum(hn, 0.0)


def _mid_body(s_ref, deg_ref, r_ref, bl_ref, g_ref, be_ref, wln_ref, wrn_ref,
              y_ref, rn_ref):
    h = _combine(s_ref, deg_ref, r_ref, bl_ref, g_ref, be_ref)
    y_ref[...] = jnp.dot(h, wln_ref[...].T, preferred_element_type=jnp.float32)
    rn_ref[...] = jnp.dot(h, wrn_ref[...].T, preferred_element_type=jnp.float32)


_tc_mid = pl.pallas_call(
    _mid_body,
    out_shape=[
        jax.ShapeDtypeStruct((_N, _H), jnp.float32),
        jax.ShapeDtypeStruct((_N, _H), jnp.float32),
    ],
)


def _final_body(s_ref, deg_ref, r_ref, bl_ref, g_ref, be_ref,
                wm1_ref, bm1_ref, wm2_ref, bm2_ref, out_ref):
    h = _combine(s_ref, deg_ref, r_ref, bl_ref, g_ref, be_ref)
    z = jnp.dot(h, wm1_ref[...].T, preferred_element_type=jnp.float32)
    z = z + bm1_ref[...]
    z2 = jnp.dot(z, wm2_ref[...].T, preferred_element_type=jnp.float32)
    z2 = z2 + bm2_ref[...]
    m = jnp.max(z2, axis=1, keepdims=True)
    lse = jnp.log(jnp.sum(jnp.exp(z2 - m), axis=1, keepdims=True)) + m
    out_ref[...] = z2 - lse


_tc_final = pl.pallas_call(
    _final_body,
    out_shape=jax.ShapeDtypeStruct((_N, _C), jnp.float32),
)


def kernel(x, edge_index, batch, Wl1, bl1, Wr1, Wl2, bl2, Wr2, Wl3, bl3, Wr3,
           g1, be1, g2, be2, g3, be3, Wm1, bm1, Wm2, bm2):
    del batch
    src = edge_index[0].astype(jnp.int32)
    dst = edge_index[1].astype(jnp.int32)

    bl1r, bl2r, bl3r = (b.reshape(1, _H) for b in (bl1, bl2, bl3))
    g1r, g2r, g3r = (g.reshape(1, _H) for g in (g1, g2, g3))
    be1r, be2r, be3r = (b.reshape(1, _H) for b in (be1, be2, be3))
    bm1r = bm1.reshape(1, _H)
    bm2r = bm2.reshape(1, _C)

    y1, r1 = _tc_pre(x, Wl1, Wr1)
    deg = _sc_deg()(dst)
    s1 = _sc_agg()(y1, src, dst)
    y2, r2 = _tc_mid(s1, deg, r1, bl1r, g1r, be1r, Wl2, Wr2)
    s2 = _sc_agg()(y2, src, dst)
    y3, r3 = _tc_mid(s2, deg, r2, bl2r, g2r, be2r, Wl3, Wr3)
    s3 = _sc_agg()(y3, src, dst)
    out = _tc_final(s3, deg, r3, bl3r, g3r, be3r, Wm1, bm1r, Wm2, bm2r)
    return out


# speedup vs baseline: 9.6207x; 1.0034x over previous
import functools

import jax
import jax.numpy as jnp
from jax import lax
from jax.experimental import pallas as pl
from jax.experimental.pallas import tpu as pltpu
from jax.experimental.pallas import tpu_sc as plsc

_N = 10000
_E = 320000
_H = 128
_C = 40
_EPS = 1e-5

_NC = 2
_NS = 16
_NW = _NC * _NS
_EPT = _E // _NW
_CH = 64
_NCHUNK = _EPT // _CH
_CHT = _EPT - _NCHUNK * _CH
_RPB = 624
_RTAIL = _N - _NS * _RPB
_DW = 16

def _mesh():
    return plsc.VectorSubcoreMesh(
        core_axis_name="c", subcore_axis_name="s",
        num_cores=_NC, num_subcores=_NS,
    )


def _zero_rows(ref, nrows, width):
    zv = jnp.zeros((16,), jnp.float32)

    def body(i, carry):
        for j in range(width // 16):
            ref[i, pl.ds(j * 16, 16)] = zv
        return carry

    lax.fori_loop(0, nrows, body, 0)


def _fill_ones(ref, nrows, width):
    ov = jnp.ones((16,), jnp.float32)

    def body(i, carry):
        for j in range(width // 16):
            ref[i, pl.ds(j * 16, 16)] = ov
        return carry

    lax.fori_loop(0, nrows, body, 0)


def _stripe_chunks(total, step):
    chunks = [(k * step, step) for k in range(total // step)]
    if total % step:
        chunks.append((total - total % step, total % step))
    return chunks


def _drain_stripe(src_sh, dst_hbm, c, s, chunk_rows):
    r0 = pl.multiple_of(s * _RPB, 8)
    for off, sz in _stripe_chunks(_RPB, chunk_rows):
        pltpu.sync_copy(src_sh.at[pl.ds(r0 + off, sz)],
                        dst_hbm.at[c, pl.ds(r0 + off, sz)])

    @pl.when(s == _NS - 1)
    def _tail():
        pltpu.sync_copy(src_sh.at[pl.ds(_NS * _RPB, _RTAIL)],
                        dst_hbm.at[c, pl.ds(_NS * _RPB, _RTAIL)])


def _zero_stripe(acc_sh, s, zbuf):
    r0 = pl.multiple_of(s * _RPB, 8)
    for off, sz in _stripe_chunks(_RPB, zbuf.shape[0]):
        pltpu.sync_copy(zbuf.at[pl.ds(0, sz)],
                        acc_sh.at[pl.ds(r0 + off, sz)])

    @pl.when(s == _NS - 1)
    def _tail():
        pltpu.sync_copy(zbuf.at[pl.ds(0, _RTAIL)],
                        acc_sh.at[pl.ds(_NS * _RPB, _RTAIL)])


_DNBUF = 12
_NBUF = 6
assert _NCHUNK % _NBUF == 0 and _NCHUNK % _DNBUF == 0


def _sc_deg_body(dst_hbm, deg_hbm, dst_vs, dtail_v, ones_v, zbuf, deg_sh,
                 isems, ssems):
    c = lax.axis_index("c")
    s = lax.axis_index("s")
    t = c * _NS + s

    _zero_rows(zbuf, _CH, _H)
    _fill_ones(ones_v, _CH, _H)
    _zero_stripe(deg_sh, s, zbuf)
    plsc.subcore_barrier()

    e0 = t * _EPT

    def wait_scatter(b):
        pltpu.make_async_copy(ones_v, deg_sh.at[dst_vs[b]], ssems[b]).wait()

    def group(i, carry):
        j0 = i * _DNBUF
        ids = []
        for b in range(_DNBUF):
            @pl.when(i > 0)
            def _w(b=b):
                wait_scatter(b)

            base = pl.multiple_of(e0 + (j0 + b) * _CH, 8)
            ids.append(pltpu.async_copy(dst_hbm.at[pl.ds(base, _CH)],
                                        dst_vs[b], isems[b]))
        for b in range(_DNBUF):
            ids[b].wait()
            pltpu.async_copy(ones_v, deg_sh.at[dst_vs[b]], ssems[b],
                             add=True)
        return carry

    lax.fori_loop(0, _NCHUNK // _DNBUF, group, 0)
    for b in range(_DNBUF):
        wait_scatter(b)
    base = pl.multiple_of(e0 + _NCHUNK * _CH, 8)
    ones_t = ones_v.at[pl.ds(0, _CHT)]
    pltpu.async_copy(dst_hbm.at[pl.ds(base, _CHT)], dtail_v, isems[0]).wait()
    pltpu.async_copy(ones_t, deg_sh.at[dtail_v], ssems[0], add=True).wait()
    plsc.subcore_barrier()

    _drain_stripe(deg_sh, deg_hbm, c, s, _CH)


def _sc_agg_body(y_hbm, src_hbm, dst_hbm, out_hbm,
                 src_vs, dst_vs, rows, stail_v, dtail_v, acc_sh,
                 isems, gsems, ssems):
    c = lax.axis_index("c")
    s = lax.axis_index("s")
    t = c * _NS + s

    _zero_rows(rows[0], _CH, _H)
    _zero_stripe(acc_sh, s, rows[0])
    plsc.subcore_barrier()

    e0 = t * _EPT

    def wait_scatter(b):
        pltpu.make_async_copy(rows[b], acc_sh.at[dst_vs[b]], ssems[b]).wait()

    def group(i, carry):
        j0 = i * _NBUF
        ids, gds = [], []
        for b in range(_NBUF):
            @pl.when(i > 0)
            def _w(b=b):
                wait_scatter(b)

            base = pl.multiple_of(e0 + (j0 + b) * _CH, 8)
            i0 = pltpu.async_copy(src_hbm.at[pl.ds(base, _CH)], src_vs[b],
                                  isems[b])
            i1 = pltpu.async_copy(dst_hbm.at[pl.ds(base, _CH)], dst_vs[b],
                                  isems[b])
            ids.append((i0, i1))
        for b in range(_NBUF):
            ids[b][0].wait()
            gds.append(pltpu.async_copy(y_hbm.at[src_vs[b]], rows[b],
                                        gsems[b]))
        for b in range(_NBUF):
            gds[b].wait()
            ids[b][1].wait()
            pltpu.async_copy(rows[b], acc_sh.at[dst_vs[b]], ssems[b],
                             add=True)
        return carry

    lax.fori_loop(0, _NCHUNK // _NBUF, group, 0)
    for b in range(_NBUF):
        wait_scatter(b)
    base = pl.multiple_of(e0 + _NCHUNK * _CH, 8)
    rows_t = rows[0].at[pl.ds(0, _CHT)]
    i0 = pltpu.async_copy(src_hbm.at[pl.ds(base, _CHT)], stail_v, isems[0])
    i1 = pltpu.async_copy(dst_hbm.at[pl.ds(base, _CHT)], dtail_v, isems[0])
    i0.wait()
    pltpu.async_copy(y_hbm.at[stail_v], rows_t, gsems[0]).wait()
    i1.wait()
    pltpu.async_copy(rows_t, acc_sh.at[dtail_v], ssems[0], add=True).wait()
    plsc.subcore_barrier()

    _drain_stripe(acc_sh, out_hbm, c, s, _CH)


@functools.cache
def _sc_deg():
    return pl.kernel(
        _sc_deg_body,
        out_type=jax.ShapeDtypeStruct((_NC, _N, _H), jnp.float32),
        mesh=_mesh(),
        scratch_types=[
            [pltpu.VMEM((_CH,), jnp.int32) for _ in range(_DNBUF)],
            pltpu.VMEM((_CHT,), jnp.int32),
            pltpu.VMEM((_CH, _H), jnp.float32),
            pltpu.VMEM((_CH, _H), jnp.float32),
            pltpu.VMEM_SHARED((_N, _H), jnp.float32),
            [pltpu.SemaphoreType.DMA for _ in range(_DNBUF)],
            [pltpu.SemaphoreType.DMA for _ in range(_DNBUF)],
        ],
    )


@functools.cache
def _sc_agg():
    return pl.kernel(
        _sc_agg_body,
        out_type=jax.ShapeDtypeStruct((_NC, _N, _H), jnp.float32),
        mesh=_mesh(),
        scratch_types=[
            [pltpu.VMEM((_CH,), jnp.int32) for _ in range(_NBUF)],
            [pltpu.VMEM((_CH,), jnp.int32) for _ in range(_NBUF)],
            [pltpu.VMEM((_CH, _H), jnp.float32) for _ in range(_NBUF)],
            pltpu.VMEM((_CHT,), jnp.int32),
            pltpu.VMEM((_CHT,), jnp.int32),
            pltpu.VMEM_SHARED((_N, _H), jnp.float32),
            [pltpu.SemaphoreType.DMA for _ in range(_NBUF)],
            [pltpu.SemaphoreType.DMA for _ in range(_NBUF)],
            [pltpu.SemaphoreType.DMA for _ in range(_NBUF)],
        ],
    )


def _pre_body(x_ref, wl_ref, wr_ref, y_ref, r_ref):
    x = x_ref[...]
    y_ref[...] = jnp.dot(x, wl_ref[...].T, preferred_element_type=jnp.float32)
    r_ref[...] = jnp.dot(x, wr_ref[...].T, preferred_element_type=jnp.float32)


_tc_pre = pl.pallas_call(
    _pre_body,
    out_shape=[
        jax.ShapeDtypeStruct((_N, _H), jnp.float32),
        jax.ShapeDtypeStruct((_N, _H), jnp.float32),
    ],
)


def _combine(s_ref, deg_ref, r_ref, bl_ref, g_ref, be_ref):
    ssum = s_ref[0] + s_ref[1]
    degw = deg_ref[0] + deg_ref[1]
    deg = jnp.sum(degw, axis=1, keepdims=True) * (1.0 / _H)
    agg = ssum / jnp.maximum(deg, 1.0)
    h = agg + bl_ref[...] + r_ref[...]
    mean = jnp.mean(h, axis=0, keepdims=True)
    var = jnp.mean((h - mean) ** 2, axis=0, keepdims=True)
    hn = g_ref[...] * (h - mean) * lax.rsqrt(var + _EPS) + be_ref[...]
    return jnp.maximum(hn, 0.0)


def _mid_body(s_ref, deg_ref, r_ref, bl_ref, g_ref, be_ref, wln_ref, wrn_ref,
              y_ref, rn_ref):
    h = _combine(s_ref, deg_ref, r_ref, bl_ref, g_ref, be_ref)
    y_ref[...] = jnp.dot(h, wln_ref[...].T, preferred_element_type=jnp.float32)
    rn_ref[...] = jnp.dot(h, wrn_ref[...].T, preferred_element_type=jnp.float32)


_tc_mid = pl.pallas_call(
    _mid_body,
    out_shape=[
        jax.ShapeDtypeStruct((_N, _H), jnp.float32),
        jax.ShapeDtypeStruct((_N, _H), jnp.float32),
    ],
)


def _final_body(s_ref, deg_ref, r_ref, bl_ref, g_ref, be_ref,
                wm1_ref, bm1_ref, wm2_ref, bm2_ref, out_ref):
    h = _combine(s_ref, deg_ref, r_ref, bl_ref, g_ref, be_ref)
    z = jnp.dot(h, wm1_ref[...].T, preferred_element_type=jnp.float32)
    z = z + bm1_ref[...]
    z2 = jnp.dot(z, wm2_ref[...].T, preferred_element_type=jnp.float32)
    z2 = z2 + bm2_ref[...]
    m = jnp.max(z2, axis=1, keepdims=True)
    lse = jnp.log(jnp.sum(jnp.exp(z2 - m), axis=1, keepdims=True)) + m
    out_ref[...] = z2 - lse


_tc_final = pl.pallas_call(
    _final_body,
    out_shape=jax.ShapeDtypeStruct((_N, _C), jnp.float32),
)


def kernel(x, edge_index, batch, Wl1, bl1, Wr1, Wl2, bl2, Wr2, Wl3, bl3, Wr3,
           g1, be1, g2, be2, g3, be3, Wm1, bm1, Wm2, bm2):
    del batch
    src = edge_index[0].astype(jnp.int32)
    dst = edge_index[1].astype(jnp.int32)

    bl1r, bl2r, bl3r = (b.reshape(1, _H) for b in (bl1, bl2, bl3))
    g1r, g2r, g3r = (g.reshape(1, _H) for g in (g1, g2, g3))
    be1r, be2r, be3r = (b.reshape(1, _H) for b in (be1, be2, be3))
    bm1r = bm1.reshape(1, _H)
    bm2r = bm2.reshape(1, _C)

    y1, r1 = _tc_pre(x, Wl1, Wr1)
    deg = _sc_deg()(dst)
    s1 = _sc_agg()(y1, src, dst)
    y2, r2 = _tc_mid(s1, deg, r1, bl1r, g1r, be1r, Wl2, Wr2)
    s2 = _sc_agg()(y2, src, dst)
    y3, r3 = _tc_mid(s2, deg, r2, bl2r, g2r, be2r, Wl3, Wr3)
    s3 = _sc_agg()(y3, src, dst)
    out = _tc_final(s3, deg, r3, bl3r, g3r, be3r, Wm1, bm1r, Wm2, bm2r)
    return out


# speedup vs baseline: 9.6386x; 1.0019x over previous
import functools

import jax
import jax.numpy as jnp
from jax import lax
from jax.experimental import pallas as pl
from jax.experimental.pallas import tpu as pltpu
from jax.experimental.pallas import tpu_sc as plsc

_N = 10000
_E = 320000
_H = 128
_C = 40
_EPS = 1e-5

_NC = 2
_NS = 16
_NW = _NC * _NS
_EPT = _E // _NW
_CH = 64
_NCHUNK = _EPT // _CH
_CHT = _EPT - _NCHUNK * _CH
_RPB = 624
_RTAIL = _N - _NS * _RPB

def _mesh():
    return plsc.VectorSubcoreMesh(
        core_axis_name="c", subcore_axis_name="s",
        num_cores=_NC, num_subcores=_NS,
    )


def _zero_rows(ref, nrows, width):
    zv = jnp.zeros((16,), jnp.float32)

    def body(i, carry):
        for j in range(width // 16):
            ref[i, pl.ds(j * 16, 16)] = zv
        return carry

    lax.fori_loop(0, nrows, body, 0)


def _fill_ones(ref, nrows, width):
    ov = jnp.ones((16,), jnp.float32)

    def body(i, carry):
        for j in range(width // 16):
            ref[i, pl.ds(j * 16, 16)] = ov
        return carry

    lax.fori_loop(0, nrows, body, 0)


def _stripe_chunks(total, step):
    chunks = [(k * step, step) for k in range(total // step)]
    if total % step:
        chunks.append((total - total % step, total % step))
    return chunks


def _drain_stripe(src_sh, dst_hbm, c, s, chunk_rows):
    r0 = pl.multiple_of(s * _RPB, 8)
    for off, sz in _stripe_chunks(_RPB, chunk_rows):
        pltpu.sync_copy(src_sh.at[pl.ds(r0 + off, sz)],
                        dst_hbm.at[c, pl.ds(r0 + off, sz)])

    @pl.when(s == _NS - 1)
    def _tail():
        pltpu.sync_copy(src_sh.at[pl.ds(_NS * _RPB, _RTAIL)],
                        dst_hbm.at[c, pl.ds(_NS * _RPB, _RTAIL)])


def _zero_stripe(acc_sh, s, zbuf):
    r0 = pl.multiple_of(s * _RPB, 8)
    for off, sz in _stripe_chunks(_RPB, zbuf.shape[0]):
        pltpu.sync_copy(zbuf.at[pl.ds(0, sz)],
                        acc_sh.at[pl.ds(r0 + off, sz)])

    @pl.when(s == _NS - 1)
    def _tail():
        pltpu.sync_copy(zbuf.at[pl.ds(0, _RTAIL)],
                        acc_sh.at[pl.ds(_NS * _RPB, _RTAIL)])


_DNBUF = 12
_NBUF = 6
assert _NCHUNK % _NBUF == 0 and _NCHUNK % _DNBUF == 0


def _sc_deg_body(dst_hbm, deg_hbm, dst_vs, dtail_v, ones_v, zbuf, deg_sh,
                 isems, ssems):
    c = lax.axis_index("c")
    s = lax.axis_index("s")
    t = c * _NS + s

    _zero_rows(zbuf, _CH, _H)
    _fill_ones(ones_v, _CH, _H)
    _zero_stripe(deg_sh, s, zbuf)
    plsc.subcore_barrier()

    e0 = t * _EPT

    def wait_scatter(b):
        pltpu.make_async_copy(ones_v, deg_sh.at[dst_vs[b]], ssems[b]).wait()

    def group(i, carry):
        j0 = i * _DNBUF
        ids = []
        for b in range(_DNBUF):
            @pl.when(i > 0)
            def _w(b=b):
                wait_scatter(b)

            base = pl.multiple_of(e0 + (j0 + b) * _CH, 8)
            ids.append(pltpu.async_copy(dst_hbm.at[pl.ds(base, _CH)],
                                        dst_vs[b], isems[b]))
        for b in range(_DNBUF):
            ids[b].wait()
            pltpu.async_copy(ones_v, deg_sh.at[dst_vs[b]], ssems[b],
                             add=True)
        return carry

    lax.fori_loop(0, _NCHUNK // _DNBUF, group, 0)
    for b in range(_DNBUF):
        wait_scatter(b)
    base = pl.multiple_of(e0 + _NCHUNK * _CH, 8)
    ones_t = ones_v.at[pl.ds(0, _CHT)]
    pltpu.async_copy(dst_hbm.at[pl.ds(base, _CHT)], dtail_v, isems[0]).wait()
    pltpu.async_copy(ones_t, deg_sh.at[dtail_v], ssems[0], add=True).wait()
    plsc.subcore_barrier()

    _drain_stripe(deg_sh, deg_hbm, c, s, _CH)


def _sc_agg_body(y_hbm, src_hbm, dst_hbm, out_hbm,
                 src_vs, dst_vs, rows, stail_v, dtail_v, acc_sh,
                 isems, gsems, ssems):
    c = lax.axis_index("c")
    s = lax.axis_index("s")
    t = c * _NS + s

    _zero_rows(rows[0], _CH, _H)
    _zero_stripe(acc_sh, s, rows[0])
    plsc.subcore_barrier()

    e0 = t * _EPT

    def wait_scatter(b):
        pltpu.make_async_copy(rows[b], acc_sh.at[dst_vs[b]], ssems[b]).wait()

    def group(i, carry):
        j0 = i * _NBUF
        ids, gds = [], []
        for b in range(_NBUF):
            @pl.when(i > 0)
            def _w(b=b):
                wait_scatter(b)

            base = pl.multiple_of(e0 + (j0 + b) * _CH, 8)
            i0 = pltpu.async_copy(src_hbm.at[pl.ds(base, _CH)], src_vs[b],
                                  isems[b])
            i1 = pltpu.async_copy(dst_hbm.at[pl.ds(base, _CH)], dst_vs[b],
                                  isems[b])
            ids.append((i0, i1))
        for b in range(_NBUF):
            ids[b][0].wait()
            gds.append(pltpu.async_copy(y_hbm.at[src_vs[b]], rows[b],
                                        gsems[b]))
        for b in range(_NBUF):
            gds[b].wait()
            ids[b][1].wait()
            pltpu.async_copy(rows[b], acc_sh.at[dst_vs[b]], ssems[b],
                             add=True)
        return carry

    lax.fori_loop(0, _NCHUNK // _NBUF, group, 0)
    for b in range(_NBUF):
        wait_scatter(b)
    base = pl.multiple_of(e0 + _NCHUNK * _CH, 8)
    rows_t = rows[0].at[pl.ds(0, _CHT)]
    i0 = pltpu.async_copy(src_hbm.at[pl.ds(base, _CHT)], stail_v, isems[0])
    i1 = pltpu.async_copy(dst_hbm.at[pl.ds(base, _CHT)], dtail_v, isems[0])
    i0.wait()
    pltpu.async_copy(y_hbm.at[stail_v], rows_t, gsems[0]).wait()
    i1.wait()
    pltpu.async_copy(rows_t, acc_sh.at[dtail_v], ssems[0], add=True).wait()
    plsc.subcore_barrier()

    _drain_stripe(acc_sh, out_hbm, c, s, _CH)


@functools.cache
def _sc_deg():
    return pl.kernel(
        _sc_deg_body,
        out_type=jax.ShapeDtypeStruct((_NC, _N, _H), jnp.float32),
        mesh=_mesh(),
        scratch_types=[
            [pltpu.VMEM((_CH,), jnp.int32) for _ in range(_DNBUF)],
            pltpu.VMEM((_CHT,), jnp.int32),
            pltpu.VMEM((_CH, _H), jnp.float32),
            pltpu.VMEM((_CH, _H), jnp.float32),
            pltpu.VMEM_SHARED((_N, _H), jnp.float32),
            [pltpu.SemaphoreType.DMA for _ in range(_DNBUF)],
            [pltpu.SemaphoreType.DMA for _ in range(_DNBUF)],
        ],
    )


@functools.cache
def _sc_agg():
    return pl.kernel(
        _sc_agg_body,
        out_type=jax.ShapeDtypeStruct((_NC, _N, _H), jnp.float32),
        mesh=_mesh(),
        scratch_types=[
            [pltpu.VMEM((_CH,), jnp.int32) for _ in range(_NBUF)],
            [pltpu.VMEM((_CH,), jnp.int32) for _ in range(_NBUF)],
            [pltpu.VMEM((_CH, _H), jnp.float32) for _ in range(_NBUF)],
            pltpu.VMEM((_CHT,), jnp.int32),
            pltpu.VMEM((_CHT,), jnp.int32),
            pltpu.VMEM_SHARED((_N, _H), jnp.float32),
            [pltpu.SemaphoreType.DMA for _ in range(_NBUF)],
            [pltpu.SemaphoreType.DMA for _ in range(_NBUF)],
            [pltpu.SemaphoreType.DMA for _ in range(_NBUF)],
        ],
    )


def _pre_body(x_ref, wl_ref, wr_ref, y_ref, r_ref):
    x = x_ref[...]
    y_ref[...] = jnp.dot(x, wl_ref[...].T, preferred_element_type=jnp.float32)
    r_ref[...] = jnp.dot(x, wr_ref[...].T, preferred_element_type=jnp.float32)


_tc_pre = pl.pallas_call(
    _pre_body,
    out_shape=[
        jax.ShapeDtypeStruct((_N, _H), jnp.float32),
        jax.ShapeDtypeStruct((_N, _H), jnp.float32),
    ],
)


def _combine(s_ref, deg_ref, r_ref, bl_ref, g_ref, be_ref):
    ssum = s_ref[0] + s_ref[1]
    degw = deg_ref[0] + deg_ref[1]
    deg = jnp.sum(degw, axis=1, keepdims=True) * (1.0 / _H)
    agg = ssum / jnp.maximum(deg, 1.0)
    h = agg + bl_ref[...] + r_ref[...]
    mean = jnp.mean(h, axis=0, keepdims=True)
    var = jnp.mean((h - mean) ** 2, axis=0, keepdims=True)
    hn = g_ref[...] * (h - mean) * lax.rsqrt(var + _EPS) + be_ref[...]
    return jnp.maximum(hn, 0.0)


def _mid_body(s_ref, deg_ref, r_ref, bl_ref, g_ref, be_ref, wln_ref, wrn_ref,
              y_ref, rn_ref):
    h = _combine(s_ref, deg_ref, r_ref, bl_ref, g_ref, be_ref)
    y_ref[...] = jnp.dot(h, wln_ref[...].T, preferred_element_type=jnp.float32)
    rn_ref[...] = jnp.dot(h, wrn_ref[...].T, preferred_element_type=jnp.float32)


_tc_mid = pl.pallas_call(
    _mid_body,
    out_shape=[
        jax.ShapeDtypeStruct((_N, _H), jnp.float32),
        jax.ShapeDtypeStruct((_N, _H), jnp.float32),
    ],
)


def _final_body(s_ref, deg_ref, r_ref, bl_ref, g_ref, be_ref,
                wm1_ref, bm1_ref, wm2_ref, bm2_ref, out_ref):
    h = _combine(s_ref, deg_ref, r_ref, bl_ref, g_ref, be_ref)
    z = jnp.dot(h, wm1_ref[...].T, preferred_element_type=jnp.float32)
    z = z + bm1_ref[...]
    z2 = jnp.dot(z, wm2_ref[...].T, preferred_element_type=jnp.float32)
    z2 = z2 + bm2_ref[...]
    m = jnp.max(z2, axis=1, keepdims=True)
    lse = jnp.log(jnp.sum(jnp.exp(z2 - m), axis=1, keepdims=True)) + m
    out_ref[...] = z2 - lse


_tc_final = pl.pallas_call(
    _final_body,
    out_shape=jax.ShapeDtypeStruct((_N, _C), jnp.float32),
)


def kernel(x, edge_index, batch, Wl1, bl1, Wr1, Wl2, bl2, Wr2, Wl3, bl3, Wr3,
           g1, be1, g2, be2, g3, be3, Wm1, bm1, Wm2, bm2):
    del batch
    src = edge_index[0].astype(jnp.int32)
    dst = edge_index[1].astype(jnp.int32)

    bl1r, bl2r, bl3r = (b.reshape(1, _H) for b in (bl1, bl2, bl3))
    g1r, g2r, g3r = (g.reshape(1, _H) for g in (g1, g2, g3))
    be1r, be2r, be3r = (b.reshape(1, _H) for b in (be1, be2, be3))
    bm1r = bm1.reshape(1, _H)
    bm2r = bm2.reshape(1, _C)

    y1, r1 = _tc_pre(x, Wl1, Wr1)
    deg = _sc_deg()(dst)
    s1 = _sc_agg()(y1, src, dst)
    y2, r2 = _tc_mid(s1, deg, r1, bl1r, g1r, be1r, Wl2, Wr2)
    s2 = _sc_agg()(y2, src, dst)
    y3, r3 = _tc_mid(s2, deg, r2, bl2r, g2r, be2r, Wl3, Wr3)
    s3 = _sc_agg()(y3, src, dst)
    out = _tc_final(s3, deg, r3, bl3r, g3r, be3r, Wm1, bm1r, Wm2, bm2r)
    return out
